# baseline (speedup 1.0000x reference)
import functools

import jax
import jax.numpy as jnp
from jax import lax
from jax.experimental import pallas as pl
from jax.experimental.pallas import tpu as pltpu
from jax.experimental.pallas import tpu_sc as plsc

D = 128
N = 10000
E = 320000
NC = 2
NS = 16
NW = NC * NS
E_W = E // NW
E_BLK = 80
N_BLK = E_W // E_BLK


def _dot_body(z_hbm, u_hbm, v_hbm, out_hbm,
              uidx_v, vidx_v, zu0, zv0, zu1, zv1, zu2, zv2, zu3, zv3,
              out_v, z_sh, s0, s1, s2, s3):
    sid = lax.axis_index("s")
    wid = sid * NC + lax.axis_index("c")
    base = wid * E_W

    @pl.when(sid == 0)
    def _():
        pltpu.sync_copy(z_hbm, z_sh)

    pltpu.sync_copy(u_hbm.at[pl.ds(base, E_W)], uidx_v)
    pltpu.sync_copy(v_hbm.at[pl.ds(base, E_W)], vidx_v)
    plsc.subcore_barrier()

    def copies(b, zu, zv, sem):
        off = b * E_BLK
        cu = pltpu.make_async_copy(
            z_sh.at[uidx_v.at[pl.ds(off, E_BLK)]], zu, sem)
        cv = pltpu.make_async_copy(
            z_sh.at[vidx_v.at[pl.ds(off, E_BLK)]], zv, sem)
        return cu, cv

    def start(b, zu, zv, sem):
        cu, cv = copies(b, zu, zv, sem)
        cu.start()
        cv.start()

    def wait(b, zu, zv, sem):
        cu, cv = copies(b, zu, zv, sem)
        cu.wait()
        cv.wait()

    lane = lax.iota(jnp.int32, 16)

    def compute(b, zu, zv):
        def group(g, c):
            res = jnp.zeros((16,), jnp.float32)
            for j in range(16):
                e = g * 16 + j
                acc = None
                for q in range(D // 32):
                    au = plsc.bitcast(zu[e, pl.ds(q * 16, 16)], jnp.bfloat16)
                    av = plsc.bitcast(zv[e, pl.ds(q * 16, 16)], jnp.bfloat16)
                    p0, p1 = plsc.unpack(au * av,
                                         format=plsc.PackFormat.INTERLEAVED)
                    t = p0 + p1
                    acc = t if acc is None else acc + t
                res = jnp.where(lane == j, jnp.sum(acc), res)
            out_v[pl.ds(b * E_BLK + g * 16, 16)] = res
            return c

        lax.fori_loop(0, E_BLK // 16, group, 0, unroll=False)

    bufs = ((zu0, zv0, s0), (zu1, zv1, s1), (zu2, zv2, s2), (zu3, zv3, s3))

    start(0, *bufs[0])
    start(1, *bufs[1])
    start(2, *bufs[2])

    def quad(i, c):
        b = 4 * i
        for k in range(4):
            start(b + k + 3, *bufs[(k + 3) % 4])
            wait(b + k, *bufs[k])
            compute(b + k, bufs[k][0], bufs[k][1])
        return c

    lax.fori_loop(0, (N_BLK - 5) // 4, quad, 0, unroll=False)

    t = N_BLK - 5
    start(t + 3, *bufs[3])
    wait(t, *bufs[0])
    compute(t, zu0, zv0)
    start(t + 4, *bufs[0])
    wait(t + 1, *bufs[1])
    compute(t + 1, zu1, zv1)
    wait(t + 2, *bufs[2])
    compute(t + 2, zu2, zv2)
    wait(t + 3, *bufs[3])
    compute(t + 3, zu3, zv3)
    wait(t + 4, *bufs[0])
    compute(t + 4, zu0, zv0)

    pltpu.sync_copy(out_v, out_hbm.at[pl.ds(base, E_W)])


@functools.partial(jax.jit, donate_argnums=())
def _dot_sc(zbf, u, v):
    mesh = plsc.VectorSubcoreMesh(core_axis_name="c", subcore_axis_name="s")
    return pl.kernel(
        _dot_body,
        mesh=mesh,
        compiler_params=pltpu.CompilerParams(
            needs_layout_passes=False, use_tc_tiling_on_sc=False),
        out_type=jax.ShapeDtypeStruct((E,), jnp.float32),
        scratch_types=[
            pltpu.VMEM((E_W,), jnp.int32),
            pltpu.VMEM((E_W,), jnp.int32),
            pltpu.VMEM((E_BLK, D // 2), jnp.int32),
            pltpu.VMEM((E_BLK, D // 2), jnp.int32),
            pltpu.VMEM((E_BLK, D // 2), jnp.int32),
            pltpu.VMEM((E_BLK, D // 2), jnp.int32),
            pltpu.VMEM((E_BLK, D // 2), jnp.int32),
            pltpu.VMEM((E_BLK, D // 2), jnp.int32),
            pltpu.VMEM((E_BLK, D // 2), jnp.int32),
            pltpu.VMEM((E_BLK, D // 2), jnp.int32),
            pltpu.VMEM((E_W,), jnp.float32),
            pltpu.VMEM_SHARED((N, D // 2), jnp.int32),
            pltpu.SemaphoreType.DMA,
            pltpu.SemaphoreType.DMA,
            pltpu.SemaphoreType.DMA,
            pltpu.SemaphoreType.DMA,
        ],
    )(zbf, u, v)


def kernel(z, edge_index):
    u = edge_index[0].astype(jnp.int32)
    v = edge_index[1].astype(jnp.int32)
    zpacked = jax.lax.bitcast_convert_type(
        z.astype(jnp.bfloat16).reshape(N, D // 2, 2), jnp.int32)
    return _dot_sc(zpacked, u, v)


# speedup vs baseline: 1.2468x; 1.2468x over previous
import functools

import jax
import jax.numpy as jnp
from jax import lax
from jax.experimental import pallas as pl
from jax.experimental.pallas import tpu as pltpu
from jax.experimental.pallas import tpu_sc as plsc

D = 128
N = 10000
E = 320000
NC = 2
NS = 16
NW = NC * NS
E_W = E // NW
E_BLK = 80
N_BLK = E_W // E_BLK


def _dot_body(z_hbm, pidx_hbm, out_hbm,
              idx_v, zb0, zb1, out_v, z_sh, s0, s1):
    sid = lax.axis_index("s")
    wid = sid * NC + lax.axis_index("c")
    base = wid * E_W

    @pl.when(sid == 0)
    def _():
        pltpu.sync_copy(z_hbm, z_sh)

    pltpu.sync_copy(pidx_hbm.at[pl.ds(2 * base, 2 * E_W)], idx_v)
    plsc.subcore_barrier()

    def copy(b, zb, sem):
        return pltpu.make_async_copy(
            z_sh.at[idx_v.at[pl.ds(b * 2 * E_BLK, 2 * E_BLK)]], zb, sem)

    lane = lax.iota(jnp.int32, 16)

    def compute(b, zb):
        def group(g, c):
            res = jnp.zeros((16,), jnp.float32)
            for j in range(16):
                e = g * 16 + j
                acc = None
                for q in range(D // 32):
                    au = plsc.bitcast(zb[e, pl.ds(q * 16, 16)], jnp.bfloat16)
                    av = plsc.bitcast(zb[E_BLK + e, pl.ds(q * 16, 16)],
                                      jnp.bfloat16)
                    p0, p1 = plsc.unpack(au * av,
                                         format=plsc.PackFormat.INTERLEAVED)
                    t = p0 + p1
                    acc = t if acc is None else acc + t
                res = jnp.where(lane == j, jnp.sum(acc), res)
            out_v[pl.ds(b * E_BLK + g * 16, 16)] = res
            return c

        lax.fori_loop(0, E_BLK // 16, group, 0, unroll=False)

    copy(0, zb0, s0).start()

    def pair(i, c):
        b0 = 2 * i
        copy(b0 + 1, zb1, s1).start()
        copy(b0, zb0, s0).wait()
        compute(b0, zb0)
        copy(b0 + 2, zb0, s0).start()
        copy(b0 + 1, zb1, s1).wait()
        compute(b0 + 1, zb1)
        return c

    lax.fori_loop(0, N_BLK // 2, pair, 0, unroll=False)
    copy(N_BLK - 1, zb0, s0).wait()
    compute(N_BLK - 1, zb0)

    pltpu.sync_copy(out_v, out_hbm.at[pl.ds(base, E_W)])


@functools.partial(jax.jit, donate_argnums=())
def _dot_sc(zbf, pidx):
    mesh = plsc.VectorSubcoreMesh(core_axis_name="c", subcore_axis_name="s")
    return pl.kernel(
        _dot_body,
        mesh=mesh,
        compiler_params=pltpu.CompilerParams(
            needs_layout_passes=False, use_tc_tiling_on_sc=False),
        out_type=jax.ShapeDtypeStruct((E,), jnp.float32),
        scratch_types=[
            pltpu.VMEM((2 * E_W,), jnp.int32),
            pltpu.VMEM((2 * E_BLK, D // 2), jnp.int32),
            pltpu.VMEM((2 * E_BLK, D // 2), jnp.int32),
            pltpu.VMEM((E_W,), jnp.float32),
            pltpu.VMEM_SHARED((N, D // 2), jnp.int32),
            pltpu.SemaphoreType.DMA,
            pltpu.SemaphoreType.DMA,
        ],
    )(zbf, pidx)


def kernel(z, edge_index):
    uu = edge_index[0].astype(jnp.int32).reshape(NW, N_BLK, E_BLK)
    vv = edge_index[1].astype(jnp.int32).reshape(NW, N_BLK, E_BLK)
    pidx = jnp.concatenate([uu, vv], axis=2).reshape(-1)
    zpacked = jax.lax.bitcast_convert_type(
        z.astype(jnp.bfloat16).reshape(N, D // 2, 2), jnp.int32)
    return _dot_sc(zpacked, pidx)


# speedup vs baseline: 1.3446x; 1.0785x over previous
import functools

import jax
import jax.numpy as jnp
from jax import lax
from jax.experimental import pallas as pl
from jax.experimental.pallas import tpu as pltpu
from jax.experimental.pallas import tpu_sc as plsc

D = 128
N = 10000
E = 320000
NC = 2
NS = 16
NW = NC * NS
E_W = E // NW
E_BLK = 80
N_BLK = E_W // E_BLK


def _dot_body(z_hbm, u_hbm, v_hbm, out_hbm,
              uidx_v, vidx_v, zu0, zv0, zu1, zv1, out_v, z_sh, s0, s1):
    sid = lax.axis_index("s")
    wid = sid * NC + lax.axis_index("c")
    base = wid * E_W

    @pl.when(sid == 0)
    def _():
        pltpu.sync_copy(z_hbm, z_sh)

    pltpu.sync_copy(u_hbm.at[pl.ds(base, E_W)], uidx_v)
    pltpu.sync_copy(v_hbm.at[pl.ds(base, E_W)], vidx_v)
    plsc.subcore_barrier()

    def copies(b, zu, zv, sem):
        off = b * E_BLK
        cu = pltpu.make_async_copy(
            z_sh.at[uidx_v.at[pl.ds(off, E_BLK)]], zu, sem)
        cv = pltpu.make_async_copy(
            z_sh.at[vidx_v.at[pl.ds(off, E_BLK)]], zv, sem)
        return cu, cv

    def start(b, zu, zv, sem):
        cu, cv = copies(b, zu, zv, sem)
        cu.start()
        cv.start()

    def wait(b, zu, zv, sem):
        cu, cv = copies(b, zu, zv, sem)
        cu.wait()
        cv.wait()

    lane = lax.iota(jnp.int32, 16)

    def compute(b, zu, zv):
        def group(g, c):
            res = jnp.zeros((16,), jnp.float32)
            for j in range(16):
                e = g * 16 + j
                acc = None
                for q in range(D // 32):
                    au = plsc.bitcast(zu[e, pl.ds(q * 16, 16)], jnp.bfloat16)
                    av = plsc.bitcast(zv[e, pl.ds(q * 16, 16)], jnp.bfloat16)
                    p0, p1 = plsc.unpack(au * av,
                                         format=plsc.PackFormat.INTERLEAVED)
                    t = p0 + p1
                    acc = t if acc is None else acc + t
                res = jnp.where(lane == j, jnp.sum(acc), res)
            out_v[pl.ds(b * E_BLK + g * 16, 16)] = res
            return c

        lax.fori_loop(0, E_BLK // 16, group, 0, unroll=False)

    start(0, zu0, zv0, s0)

    def pair(i, c):
        b0 = 2 * i
        start(b0 + 1, zu1, zv1, s1)
        wait(b0, zu0, zv0, s0)
        compute(b0, zu0, zv0)
        start(b0 + 2, zu0, zv0, s0)
        wait(b0 + 1, zu1, zv1, s1)
        compute(b0 + 1, zu1, zv1)
        return c

    lax.fori_loop(0, N_BLK // 2, pair, 0, unroll=False)
    wait(N_BLK - 1, zu0, zv0, s0)
    compute(N_BLK - 1, zu0, zv0)

    pltpu.sync_copy(out_v, out_hbm.at[pl.ds(base, E_W)])


@functools.partial(jax.jit, donate_argnums=())
def _dot_sc(zbf, u, v):
    mesh = plsc.VectorSubcoreMesh(core_axis_name="c", subcore_axis_name="s")
    return pl.kernel(
        _dot_body,
        mesh=mesh,
        compiler_params=pltpu.CompilerParams(
            needs_layout_passes=False, use_tc_tiling_on_sc=False),
        out_type=jax.ShapeDtypeStruct((E,), jnp.float32),
        scratch_types=[
            pltpu.VMEM((E_W,), jnp.int32),
            pltpu.VMEM((E_W,), jnp.int32),
            pltpu.VMEM((E_BLK, D // 2), jnp.int32),
            pltpu.VMEM((E_BLK, D // 2), jnp.int32),
            pltpu.VMEM((E_BLK, D // 2), jnp.int32),
            pltpu.VMEM((E_BLK, D // 2), jnp.int32),
            pltpu.VMEM((E_W,), jnp.float32),
            pltpu.VMEM_SHARED((N, D // 2), jnp.int32),
            pltpu.SemaphoreType.DMA,
            pltpu.SemaphoreType.DMA,
        ],
    )(zbf, u, v)


def kernel(z, edge_index):
    u = edge_index[0].astype(jnp.int32)
    v = edge_index[1].astype(jnp.int32)
    zpacked = jax.lax.bitcast_convert_type(
        z.astype(jnp.bfloat16).reshape(N, D // 2, 2), jnp.int32)
    return _dot_sc(zpacked, u, v)
